# baseline (speedup 1.0000x reference)
import functools

import jax
import jax.numpy as jnp
from jax import lax
from jax.experimental import pallas as pl
from jax.experimental.pallas import tpu as pltpu
from jax.experimental.pallas import tpu_sc as plsc

N = 10000
E = 320000
D = 128
HE = 64
H = 128

NC = 2
NS = 16
NW = NC * NS
EPT = 10112
EPAD = NW * EPT
NP = 10240
W = 8

_mesh = plsc.VectorSubcoreMesh(core_axis_name="c", subcore_axis_name="s")
_sc_params = pltpu.CompilerParams(use_tc_tiling_on_sc=False)


@functools.partial(
    pl.kernel,
    out_type=jax.ShapeDtypeStruct((NC, 2 * NP, W), jnp.float32),
    mesh=_mesh,
    scratch_types=[
        pltpu.VMEM((2, EPT), jnp.int32),
        pltpu.VMEM((EPT, W), jnp.float32),
        pltpu.VMEM_SHARED((2 * NP, W), jnp.float32),
        pltpu.SemaphoreType.DMA,
    ],
    compiler_params=_sc_params,
)
def _k1_degrees(ec_hbm, ones_hbm, zeros_hbm, degp_hbm, idx_v, ones_v, deg_s,
                sem):
    cid = lax.axis_index("c")
    sid = lax.axis_index("s")
    wid = sid * NC + cid
    pltpu.sync_copy(ec_hbm.at[wid], idx_v)
    pltpu.sync_copy(ones_hbm, ones_v)

    @pl.when(sid == 0)
    def _():
        pltpu.sync_copy(zeros_hbm, deg_s)

    plsc.subcore_barrier()
    d0 = pltpu.async_copy(ones_v, deg_s.at[idx_v.at[0]], sem, add=True)
    d1 = pltpu.async_copy(ones_v, deg_s.at[idx_v.at[1]], sem, add=True)
    d0.wait()
    d1.wait()
    plsc.subcore_barrier()

    @pl.when(sid == 0)
    def _():
        pltpu.sync_copy(deg_s, degp_hbm.at[cid])


def _k2_body(degp_ref, x_ref, we_ref, fe_ref, wg_ref, cw_ref,
             norms_ref, v_ref):
    deg = degp_ref[0] + degp_ref[1]
    no = lax.rsqrt(jnp.maximum(deg[:NP, 0:1], 1.0))
    ni = lax.rsqrt(jnp.maximum(deg[NP:, 0:1], 1.0))
    norms_ref[...] = jnp.concatenate([no, ni], axis=1)
    wv0 = jnp.dot(we_ref[...], fe_ref[...], preferred_element_type=jnp.float32)
    wv1 = jnp.dot(wg_ref[...], cw_ref[...], preferred_element_type=jnp.float32)
    wv = jnp.concatenate([wv0, wv1], axis=1)
    u = jnp.dot(x_ref[...], wv, preferred_element_type=jnp.float32)
    v2 = u * no
    v_ref[...] = jnp.concatenate(
        [v2, jnp.zeros((NP, W - 2), jnp.float32)], axis=1)


_k2 = pl.pallas_call(
    _k2_body,
    out_shape=(
        jax.ShapeDtypeStruct((NP, 2), jnp.float32),
        jax.ShapeDtypeStruct((NP, W), jnp.float32),
    ),
)


@functools.partial(
    pl.kernel,
    out_type=jax.ShapeDtypeStruct((NC, NP, W), jnp.float32),
    mesh=_mesh,
    scratch_types=[
        pltpu.VMEM((2, EPT), jnp.int32),
        pltpu.VMEM((EPT, W), jnp.float32),
        pltpu.VMEM_SHARED((2 * NP, W), jnp.float32),
        pltpu.SemaphoreType.DMA,
        pltpu.SemaphoreType.DMA,
    ],
    compiler_params=_sc_params,
)
def _k3_edges(ec_hbm, v_hbm, zeros_hbm, aggp_hbm, idx_v, vals_v, agg_s,
              gsem, ssem):
    cid = lax.axis_index("c")
    sid = lax.axis_index("s")
    wid = sid * NC + cid
    pltpu.sync_copy(ec_hbm.at[wid], idx_v)

    @pl.when(sid == 0)
    def _():
        pltpu.sync_copy(zeros_hbm, agg_s)

    pltpu.async_copy(v_hbm.at[idx_v.at[0]], vals_v, gsem).wait()
    plsc.subcore_barrier()
    pltpu.async_copy(vals_v, agg_s.at[idx_v.at[1]], ssem, add=True).wait()
    plsc.subcore_barrier()

    @pl.when(sid == 0)
    def _():
        pltpu.sync_copy(agg_s.at[pl.ds(NP, NP)], aggp_hbm.at[cid])


def _k4_body(aggp_ref, ni_ref, be_ref, fe_ref, feb_ref, bg_ref, cw_ref,
             cb_ref, y_ref, s_ref):
    agg = aggp_ref[0] + aggp_ref[1]
    a = agg[:N]
    ni = ni_ref[...]
    cs = jnp.sum(jnp.dot(be_ref[...], fe_ref[...],
                         preferred_element_type=jnp.float32)) + jnp.sum(feb_ref[...])
    cy = jnp.sum(jnp.dot(bg_ref[...], cw_ref[...],
                         preferred_element_type=jnp.float32)) + jnp.sum(cb_ref[...])
    s_ref[...] = a[:, 0:1] * ni + cs
    y_ref[...] = a[:, 1:2] * ni + cy


_k4 = pl.pallas_call(
    _k4_body,
    out_shape=(
        jax.ShapeDtypeStruct((N, 1), jnp.float32),
        jax.ShapeDtypeStruct((N, 1), jnp.float32),
    ),
)


def kernel(x, edge_index, W_est, b_est, fc_est_W, fc_est_b, W_gnn, b_gnn,
           cls_W, cls_b):
    pad_e = jnp.full((2, EPAD - E), N, jnp.int32)
    e_pad = jnp.concatenate([edge_index, pad_e], axis=1)
    e_c = jnp.stack([e_pad[0].reshape(NW, EPT),
                     e_pad[1].reshape(NW, EPT) + NP], axis=1)
    eyeW = jnp.eye(W, dtype=jnp.float32)
    ones = jnp.tile(eyeW[0:1], (EPT, 1))
    zeros = jnp.zeros((2 * NP, W), jnp.float32)
    x_pad = jnp.pad(x, ((0, NP - N), (0, 0)))

    degp = _k1_degrees(e_c, ones, zeros)
    norms, v = _k2(degp, x_pad, W_est, fc_est_W, W_gnn, cls_W)
    aggp = _k3_edges(e_c, v, zeros)
    ni_col = norms[:N, 1:2]
    y, s = _k4(aggp, ni_col, b_est.reshape(1, HE), fc_est_W,
               fc_est_b.reshape(1, 1), b_gnn.reshape(1, H), cls_W,
               cls_b.reshape(1, 1))
    return (y, s)
